# Initial kernel scaffold; baseline (speedup 1.0000x reference)
import jax
import jax.numpy as jnp
from jax.experimental import pallas as pl


def kernel(x, adj_in, m, w1, w11, w_prime1, w_prime11, w2, w22, w_prime2, w_prime22, w_final, b_final):
    raise NotImplementedError("write your pallas kernel here")


# speedup vs baseline: 1.9215x; 1.9215x over previous
import jax
import jax.numpy as jnp
from jax.experimental import pallas as pl
from jax.experimental.pallas import tpu as pltpu

_B, _T, _N = 256, 16, 1024
_NSTAT = 10
_T_ITERS = 20
_FILTER_POS = 512.0


def _gcn_body(params_ref, adjT_ref, h0_ref, h1_ref, xs_ref, m_ref, out_ref):
    c1 = params_ref[0]
    c2 = params_ref[1]
    ca = params_ref[2]
    wf = params_ref[3]
    bf = params_ref[4]

    adjT = adjT_ref[...]
    h0 = h0_ref[...]
    h1 = h1_ref[...]

    def dot(a, b):
        return jax.lax.dot_general(
            a, b, (((1,), (0,)), ((), ())), preferred_element_type=jnp.float32)

    rs = jnp.sum(adjT, axis=0, keepdims=True)

    t0 = dot(h0, adjT)
    t1 = dot(h1, adjT)
    d = jnp.log(h0 + 1e-10) - jnp.log(h1 + 1e-10) - (c1 * t0 + c2 * t1)
    h0 = jax.nn.sigmoid(d)
    h1 = jax.nn.sigmoid(-d)

    for _ in range(_T_ITERS - 1):
        t0 = dot(h0, adjT)
        d = jnp.log(h0 + 1e-10) - jnp.log(h1 + 1e-10) - (ca * t0 + c2 * rs)
        h0 = jax.nn.sigmoid(d)
        h1 = jax.nn.sigmoid(-d)

    lane = jax.lax.broadcasted_iota(jnp.int32, (1, _N), 1).astype(jnp.float32)
    w = jax.nn.sigmoid(lane - _FILTER_POS)

    xs = xs_ref[...]
    m10 = m_ref[:, :_NSTAT, :]
    mw = m10 * w[None]
    num = jnp.sum(xs * mw, axis=2)
    den = jnp.sum(mw, axis=2) + 1e-10
    stat10 = num / den
    mean = jnp.mean(stat10, axis=1)
    std = jnp.sqrt(jnp.sum((stat10 - mean[:, None]) ** 2, axis=1) / (_NSTAT - 1))

    mlast = m_ref[:, _T - 1, :]
    mwl = mlast * w
    mean_cur = jnp.sum(h1 * mwl, axis=1) / (jnp.sum(mwl, axis=1) + 1e-10)

    z = (mean_cur - mean) / (std + 1e-6)
    out_ref[...] = jax.nn.sigmoid(z * wf + bf)


def kernel(x, adj_in, m, w1, w11, w_prime1, w_prime11, w2, w22, w_prime2,
           w_prime22, w_final, b_final):
    a = jnp.abs(w1[0])
    b = jnp.abs(w_prime1[0])
    c = jnp.abs(w_prime11[0])
    dd = jnp.abs(w11[0])
    c1 = -(a + b)
    c2 = c + dd
    ca = c1 - c2
    params = jnp.stack([c1, c2, ca, w_final[0], b_final[0],
                        jnp.float32(0), jnp.float32(0), jnp.float32(0)])

    adjT = adj_in.T
    h0 = x[:, -1, :, 0]
    h1 = x[:, -1, :, 1]
    xs = x[:, :_NSTAT, :, 1]

    out = pl.pallas_call(
        _gcn_body,
        out_shape=jax.ShapeDtypeStruct((_B,), jnp.float32),
        in_specs=[
            pl.BlockSpec(memory_space=pltpu.SMEM),
            pl.BlockSpec(memory_space=pltpu.VMEM),
            pl.BlockSpec(memory_space=pltpu.VMEM),
            pl.BlockSpec(memory_space=pltpu.VMEM),
            pl.BlockSpec(memory_space=pltpu.VMEM),
            pl.BlockSpec(memory_space=pltpu.VMEM),
        ],
        out_specs=pl.BlockSpec(memory_space=pltpu.VMEM),
        compiler_params=pltpu.CompilerParams(
            vmem_limit_bytes=100 * 1024 * 1024),
    )(params, adjT, h0, h1, xs, m)
    return out


# speedup vs baseline: 2.1162x; 1.1013x over previous
import jax
import jax.numpy as jnp
from jax.experimental import pallas as pl
from jax.experimental.pallas import tpu as pltpu

_B, _T, _N = 256, 16, 1024
_NSTAT = 10
_T_ITERS = 20
_FILTER_POS = 512.0


def _gcn_body(params_ref, adjT_ref, h0_ref, h1_ref, xs_ref, m_ref, out_ref):
    c1 = params_ref[0]
    c2 = params_ref[1]
    ca = params_ref[2]
    wf = params_ref[3]
    bf = params_ref[4]

    adjT = adjT_ref[...]
    h0 = h0_ref[...]
    h1 = h1_ref[...]

    def dot(a, b):
        return jax.lax.dot_general(
            a, b, (((1,), (0,)), ((), ())), preferred_element_type=jnp.float32)

    rs = jnp.sum(adjT, axis=0, keepdims=True)
    adjb = adjT.astype(jnp.bfloat16)

    t0 = dot(h0.astype(jnp.bfloat16), adjb)
    t1 = dot(h1.astype(jnp.bfloat16), adjb)
    d = jnp.log(h0 + 1e-10) - jnp.log(h1 + 1e-10) - (c1 * t0 + c2 * t1)

    for _ in range(_T_ITERS - 1):
        t0 = dot(jax.nn.sigmoid(d).astype(jnp.bfloat16), adjb)
        d = d - (ca * t0 + c2 * rs)
    h1 = jax.nn.sigmoid(-d)

    lane = jax.lax.broadcasted_iota(jnp.int32, (1, _N), 1).astype(jnp.float32)
    w = jax.nn.sigmoid(lane - _FILTER_POS)

    xs = xs_ref[...]
    m10 = m_ref[:, :_NSTAT, :]
    mw = m10 * w[None]
    num = jnp.sum(xs * mw, axis=2)
    den = jnp.sum(mw, axis=2) + 1e-10
    stat10 = num / den
    mean = jnp.mean(stat10, axis=1)
    std = jnp.sqrt(jnp.sum((stat10 - mean[:, None]) ** 2, axis=1) / (_NSTAT - 1))

    mlast = m_ref[:, _T - 1, :]
    mwl = mlast * w
    mean_cur = jnp.sum(h1 * mwl, axis=1) / (jnp.sum(mwl, axis=1) + 1e-10)

    z = (mean_cur - mean) / (std + 1e-6)
    out_ref[...] = jax.nn.sigmoid(z * wf + bf)


def kernel(x, adj_in, m, w1, w11, w_prime1, w_prime11, w2, w22, w_prime2,
           w_prime22, w_final, b_final):
    a = jnp.abs(w1[0])
    b = jnp.abs(w_prime1[0])
    c = jnp.abs(w_prime11[0])
    dd = jnp.abs(w11[0])
    c1 = -(a + b)
    c2 = c + dd
    ca = c1 - c2
    params = jnp.stack([c1, c2, ca, w_final[0], b_final[0],
                        jnp.float32(0), jnp.float32(0), jnp.float32(0)])

    adjT = adj_in.T
    h0 = x[:, -1, :, 0]
    h1 = x[:, -1, :, 1]
    xs = x[:, :_NSTAT, :, 1]

    out = pl.pallas_call(
        _gcn_body,
        out_shape=jax.ShapeDtypeStruct((_B,), jnp.float32),
        in_specs=[
            pl.BlockSpec(memory_space=pltpu.SMEM),
            pl.BlockSpec(memory_space=pltpu.VMEM),
            pl.BlockSpec(memory_space=pltpu.VMEM),
            pl.BlockSpec(memory_space=pltpu.VMEM),
            pl.BlockSpec(memory_space=pltpu.VMEM),
            pl.BlockSpec(memory_space=pltpu.VMEM),
        ],
        out_specs=pl.BlockSpec(memory_space=pltpu.VMEM),
        compiler_params=pltpu.CompilerParams(
            vmem_limit_bytes=100 * 1024 * 1024),
    )(params, adjT, h0, h1, xs, m)
    return out
